# Initial kernel scaffold; baseline (speedup 1.0000x reference)
import jax
import jax.numpy as jnp
from jax.experimental import pallas as pl


def kernel(hand_card_ids, hand_card_mask, hands_remaining, discards_remaining, hand_levels, rank_emb, suit_emb, proj_w, proj_b, run_ln_g, run_ln_b, hand_ln_g, hand_ln_b, level_emb):
    raise NotImplementedError("write your pallas kernel here")


# speedup vs baseline: 2.0461x; 2.0461x over previous
import functools

import jax
import jax.numpy as jnp
from jax import lax
from jax.experimental import pallas as pl
from jax.experimental.pallas import tpu as pltpu
from jax.experimental.pallas import tpu_sc as plsc

D = 128
_EPS = 1e-5

_CARD0 = 0
_MASKED = 52
_LEVEL0 = 56
_RUN0 = 72
_TROWS = 96

_NC = 2
_NS = 16
_NW = _NC * _NS
_CH = 128


def _ln_rows(x, g, b):
    mu = jnp.mean(x, axis=-1, keepdims=True)
    var = jnp.mean((x - mu) ** 2, axis=-1, keepdims=True)
    return (x - mu) / jnp.sqrt(var + _EPS) * g + b


def _table_kernel(rank_ref, suit_ref, level_ref, pwt_ref, pb_ref,
                  rg_ref, rb_ref, hg_ref, hb_ref, out_ref):
    hg = hg_ref[0:1, :]
    hb = hb_ref[0:1, :]
    card = jnp.concatenate(
        [rank_ref[:, :] + suit_ref[s:s + 1, :] for s in range(4)], axis=0)
    card_ln = _ln_rows(card, hg, hb)
    masked = jnp.broadcast_to(hb, (4, D))
    level = level_ref[:, :]
    ii = lax.broadcasted_iota(jnp.int32, (20, D), 0)
    h = (ii // 4).astype(jnp.float32)
    d = (ii % 4).astype(jnp.float32)
    x = h * pwt_ref[0:1, :] + d * pwt_ref[1:2, :] + pb_ref[0:1, :]
    run_ln = _ln_rows(x, rg_ref[0:1, :], rb_ref[0:1, :])
    pad = jnp.zeros((4, D), jnp.float32)
    out_ref[:, :] = jnp.concatenate(
        [card_ln, masked, level, run_ln, pad], axis=0)


def _build_table(rank_emb, suit_emb, level_emb, pwt, pb, rg, rb, hg, hb):
    return pl.pallas_call(
        _table_kernel,
        out_shape=jax.ShapeDtypeStruct((_TROWS, D), jnp.float32),
    )(rank_emb, suit_emb, level_emb, pwt, pb, rg, rb, hg, hb)


@functools.cache
def _make_gather(hand_rows, ctx_rows):
    hand_per = hand_rows // _NW
    ctx_per = ctx_rows // _NW
    mesh = plsc.VectorSubcoreMesh(core_axis_name="c", subcore_axis_name="s")

    @functools.partial(
        pl.kernel,
        mesh=mesh,
        out_type=(
            jax.ShapeDtypeStruct((hand_rows, D), jnp.float32),
            jax.ShapeDtypeStruct((ctx_rows, D), jnp.float32),
        ),
        scratch_types=[
            pltpu.VMEM((_CH,), jnp.int32),
            pltpu.VMEM((_CH, D), jnp.float32),
            pltpu.SemaphoreType.DMA,
        ],
    )
    def gather(table_hbm, cidx_hbm, xidx_hbm, hand_hbm, ctx_hbm,
               idx_v, rows_v, sem):
        wid = lax.axis_index("s") * _NC + lax.axis_index("c")

        def run(idx_hbm, out_hbm, rows_per_tile):
            base = wid * rows_per_tile

            def body(j, carry):
                off = base + j * _CH
                pltpu.sync_copy(idx_hbm.at[pl.ds(off, _CH)], idx_v)
                pltpu.async_copy(table_hbm.at[idx_v], rows_v, sem).wait()
                pltpu.sync_copy(rows_v, out_hbm.at[pl.ds(off, _CH)])
                return carry

            lax.fori_loop(0, rows_per_tile // _CH, body, 0)

        run(cidx_hbm, hand_hbm, hand_per)
        run(xidx_hbm, ctx_hbm, ctx_per)

    return gather


def kernel(hand_card_ids, hand_card_mask, hands_remaining, discards_remaining,
           hand_levels, rank_emb, suit_emb, proj_w, proj_b,
           run_ln_g, run_ln_b, hand_ln_g, hand_ln_b, level_emb):
    B, H = hand_card_ids.shape
    NT = hand_levels.shape[1]
    row = lambda v: v.astype(jnp.float32).reshape(1, D)
    table = _build_table(
        rank_emb.astype(jnp.float32), suit_emb.astype(jnp.float32),
        level_emb.astype(jnp.float32), proj_w.astype(jnp.float32).T,
        row(proj_b), row(run_ln_g), row(run_ln_b),
        row(hand_ln_g), row(hand_ln_b))

    ids = hand_card_ids.astype(jnp.int32)
    cidx = jnp.where(hand_card_mask, ids, _MASKED).astype(jnp.int32)
    cidx = cidx.reshape(B * H)
    run_idx = (_RUN0 + 4 * hands_remaining.astype(jnp.int32)
               + discards_remaining.astype(jnp.int32))
    xidx = jnp.concatenate(
        [hand_levels.astype(jnp.int32) + _LEVEL0, run_idx], axis=1)
    xidx = xidx.reshape(B * (NT + 1))

    hand_flat, ctx_flat = _make_gather(B * H, B * (NT + 1))(table, cidx, xidx)
    hand_toks = hand_flat.reshape(B, H, D)
    ctx_seq = ctx_flat.reshape(B, NT + 1, D)
    mask = hand_card_mask.astype(bool)
    ctx_mask = jnp.ones((B, NT + 1), dtype=bool)
    return hand_toks, mask, ctx_seq, ctx_mask


# speedup vs baseline: 2.0621x; 1.0078x over previous
import functools

import jax
import jax.numpy as jnp
from jax import lax
from jax.experimental import pallas as pl
from jax.experimental.pallas import tpu as pltpu
from jax.experimental.pallas import tpu_sc as plsc

D = 128
_EPS = 1e-5

_CARD0 = 0
_MASKED = 52
_LEVEL0 = 56
_RUN0 = 72
_TROWS = 96

_NC = 2
_NS = 16
_NW = _NC * _NS
_CH = 128


def _ln_rows(x, g, b):
    mu = jnp.mean(x, axis=-1, keepdims=True)
    var = jnp.mean((x - mu) ** 2, axis=-1, keepdims=True)
    return (x - mu) / jnp.sqrt(var + _EPS) * g + b


def _table_kernel(rank_ref, suit_ref, level_ref, pwt_ref, pb_ref,
                  rg_ref, rb_ref, hg_ref, hb_ref, out_ref):
    hg = hg_ref[0:1, :]
    hb = hb_ref[0:1, :]
    card = jnp.concatenate(
        [rank_ref[:, :] + suit_ref[s:s + 1, :] for s in range(4)], axis=0)
    card_ln = _ln_rows(card, hg, hb)
    masked = jnp.broadcast_to(hb, (4, D))
    level = level_ref[:, :]
    ii = lax.broadcasted_iota(jnp.int32, (20, D), 0)
    h = (ii // 4).astype(jnp.float32)
    d = (ii % 4).astype(jnp.float32)
    x = h * pwt_ref[0:1, :] + d * pwt_ref[1:2, :] + pb_ref[0:1, :]
    run_ln = _ln_rows(x, rg_ref[0:1, :], rb_ref[0:1, :])
    pad = jnp.zeros((4, D), jnp.float32)
    out_ref[:, :] = jnp.concatenate(
        [card_ln, masked, level, run_ln, pad], axis=0)


def _build_table(rank_emb, suit_emb, level_emb, pwt, pb, rg, rb, hg, hb):
    return pl.pallas_call(
        _table_kernel,
        out_shape=jax.ShapeDtypeStruct((_TROWS, D), jnp.float32),
    )(rank_emb, suit_emb, level_emb, pwt, pb, rg, rb, hg, hb)


@functools.cache
def _make_gather(hand_rows, ctx_rows):
    hand_ch = hand_rows // (_NW * _CH)
    ctx_ch = ctx_rows // (_NW * _CH)
    mesh = plsc.VectorSubcoreMesh(core_axis_name="c", subcore_axis_name="s")

    @functools.partial(
        pl.kernel,
        mesh=mesh,
        out_type=(
            jax.ShapeDtypeStruct((hand_rows, D), jnp.float32),
            jax.ShapeDtypeStruct((ctx_rows, D), jnp.float32),
        ),
        scratch_types=[
            pltpu.VMEM((hand_ch * _CH,), jnp.int32),
            pltpu.VMEM((ctx_ch * _CH,), jnp.int32),
            pltpu.VMEM((2, _CH, D), jnp.float32),
            pltpu.SemaphoreType.DMA,
            pltpu.SemaphoreType.DMA,
            pltpu.SemaphoreType.DMA,
            pltpu.SemaphoreType.DMA,
        ],
    )
    def gather(table_hbm, cidx_hbm, xidx_hbm, hand_hbm, ctx_hbm,
               cidx_v, xidx_v, bufs, g0, g1, o0, o1):
        wid = lax.axis_index("s") * _NC + lax.axis_index("c")
        gsem = (g0, g1)
        osem = (o0, o1)
        pltpu.sync_copy(
            cidx_hbm.at[pl.ds(pl.multiple_of(wid * (hand_ch * _CH), 8),
                              hand_ch * _CH)], cidx_v)
        pltpu.sync_copy(
            xidx_hbm.at[pl.ds(pl.multiple_of(wid * (ctx_ch * _CH), 8),
                              ctx_ch * _CH)], xidx_v)

        def run(idx_v, out_hbm, nch):
            base = wid * nch * _CH

            def out_slice(c):
                return out_hbm.at[
                    pl.ds(pl.multiple_of(base + c * _CH, 8), _CH)]

            def idx_slice(c):
                return idx_v.at[pl.ds(pl.multiple_of(c * _CH, 8), _CH)]

            def g_start(c, b):
                pltpu.async_copy(table_hbm.at[idx_slice(c)], bufs.at[b],
                                 gsem[b])

            def g_wait(c, b):
                pltpu.make_async_copy(table_hbm.at[idx_slice(c)], bufs.at[b],
                                      gsem[b]).wait()

            def s_start(c, b):
                pltpu.async_copy(bufs.at[b], out_slice(c), osem[b])

            def s_wait(c, b):
                pltpu.make_async_copy(bufs.at[b], out_slice(c),
                                      osem[b]).wait()

            g_start(0, 0)
            g_start(1, 1)
            g_wait(0, 0)
            s_start(0, 0)

            def body(g, carry):
                for u in range(2):
                    c = 1 + g * 2 + u
                    b = (1 + u) % 2
                    s_wait(c - 1, 1 - b)
                    g_start(c + 1, 1 - b)
                    g_wait(c, b)
                    s_start(c, b)
                return carry

            lax.fori_loop(0, (nch - 2) // 2, body, 0)
            c = nch - 1
            b = c % 2
            g_wait(c, b)
            s_start(c, b)
            s_wait(c - 1, 1 - b)
            s_wait(c, b)

        run(cidx_v, hand_hbm, hand_ch)
        run(xidx_v, ctx_hbm, ctx_ch)

    return gather


def kernel(hand_card_ids, hand_card_mask, hands_remaining, discards_remaining,
           hand_levels, rank_emb, suit_emb, proj_w, proj_b,
           run_ln_g, run_ln_b, hand_ln_g, hand_ln_b, level_emb):
    B, H = hand_card_ids.shape
    NT = hand_levels.shape[1]
    row = lambda v: v.astype(jnp.float32).reshape(1, D)
    table = _build_table(
        rank_emb.astype(jnp.float32), suit_emb.astype(jnp.float32),
        level_emb.astype(jnp.float32), proj_w.astype(jnp.float32).T,
        row(proj_b), row(run_ln_g), row(run_ln_b),
        row(hand_ln_g), row(hand_ln_b))

    ids = hand_card_ids.astype(jnp.int32)
    cidx = jnp.where(hand_card_mask, ids, _MASKED).astype(jnp.int32)
    cidx = cidx.reshape(B * H)
    run_idx = (_RUN0 + 4 * hands_remaining.astype(jnp.int32)
               + discards_remaining.astype(jnp.int32))
    xidx = jnp.concatenate(
        [hand_levels.astype(jnp.int32) + _LEVEL0, run_idx], axis=1)
    xidx = xidx.reshape(B * (NT + 1))

    hand_flat, ctx_flat = _make_gather(B * H, B * (NT + 1))(table, cidx, xidx)
    hand_toks = hand_flat.reshape(B, H, D)
    ctx_seq = ctx_flat.reshape(B, NT + 1, D)
    mask = hand_card_mask.astype(bool)
    ctx_mask = jnp.ones((B, NT + 1), dtype=bool)
    return hand_toks, mask, ctx_seq, ctx_mask


# speedup vs baseline: 4.7034x; 2.2809x over previous
import functools

import jax
import jax.numpy as jnp
from jax import lax
from jax.experimental import pallas as pl
from jax.experimental.pallas import tpu as pltpu
from jax.experimental.pallas import tpu_sc as plsc

D = 128
_EPS = 1e-5

_CARD0 = 0
_MASKED = 52
_LEVEL0 = 56
_RUN0 = 72
_TROWS = 96

_NC = 2
_NS = 16
_NW = _NC * _NS
_CH = 128


def _ln_rows(x, g, b):
    mu = jnp.mean(x, axis=-1, keepdims=True)
    var = jnp.mean((x - mu) ** 2, axis=-1, keepdims=True)
    return (x - mu) / jnp.sqrt(var + _EPS) * g + b


def _table_kernel(rank_ref, suit_ref, level_ref, pwt_ref, pb_ref,
                  rg_ref, rb_ref, hg_ref, hb_ref, out_ref):
    hg = hg_ref[0:1, :]
    hb = hb_ref[0:1, :]
    card = jnp.concatenate(
        [rank_ref[:, :] + suit_ref[s:s + 1, :] for s in range(4)], axis=0)
    card_ln = _ln_rows(card, hg, hb)
    masked = jnp.broadcast_to(hb, (4, D))
    level = level_ref[:, :]
    ii = lax.broadcasted_iota(jnp.int32, (20, D), 0)
    h = (ii // 4).astype(jnp.float32)
    d = (ii % 4).astype(jnp.float32)
    x = h * pwt_ref[0:1, :] + d * pwt_ref[1:2, :] + pb_ref[0:1, :]
    run_ln = _ln_rows(x, rg_ref[0:1, :], rb_ref[0:1, :])
    pad = jnp.zeros((4, D), jnp.float32)
    out_ref[:, :] = jnp.concatenate(
        [card_ln, masked, level, run_ln, pad], axis=0)


def _build_table(rank_emb, suit_emb, level_emb, pwt, pb, rg, rb, hg, hb):
    return pl.pallas_call(
        _table_kernel,
        out_shape=jax.ShapeDtypeStruct((_TROWS, D), jnp.float32),
    )(rank_emb, suit_emb, level_emb, pwt, pb, rg, rb, hg, hb)


@functools.cache
def _make_gather(hand_rows, ctx_rows):
    hand_ch = hand_rows // (_NW * _CH)
    ctx_ch = ctx_rows // (_NW * _CH)
    mesh = plsc.VectorSubcoreMesh(core_axis_name="c", subcore_axis_name="s")

    @functools.partial(
        pl.kernel,
        mesh=mesh,
        out_type=(
            jax.ShapeDtypeStruct((hand_rows, D), jnp.float32),
            jax.ShapeDtypeStruct((ctx_rows, D), jnp.float32),
        ),
        scratch_types=[
            pltpu.VMEM((hand_ch * _CH,), jnp.int32),
            pltpu.VMEM((ctx_ch * _CH,), jnp.int32),
            pltpu.VMEM((2, _CH, D), jnp.float32),
            pltpu.SemaphoreType.DMA,
            pltpu.SemaphoreType.DMA,
            pltpu.SemaphoreType.DMA,
            pltpu.SemaphoreType.DMA,
        ],
    )
    def gather(table_hbm, cidx_hbm, xidx_hbm, hand_hbm, ctx_hbm,
               cidx_v, xidx_v, bufs, g0, g1, o0, o1):
        wid = lax.axis_index("s") * _NC + lax.axis_index("c")
        gsem = (g0, g1)
        osem = (o0, o1)
        pltpu.sync_copy(
            cidx_hbm.at[pl.ds(pl.multiple_of(wid * (hand_ch * _CH), 8),
                              hand_ch * _CH)], cidx_v)
        pltpu.sync_copy(
            xidx_hbm.at[pl.ds(pl.multiple_of(wid * (ctx_ch * _CH), 8),
                              ctx_ch * _CH)], xidx_v)

        def run(idx_v, out_hbm, nch):
            base = wid * nch * _CH

            def out_slice(c):
                return out_hbm.at[
                    pl.ds(pl.multiple_of(base + c * _CH, 8), _CH)]

            def idx_slice(c):
                return idx_v.at[pl.ds(pl.multiple_of(c * _CH, 8), _CH)]

            def g_start(c, b):
                pltpu.async_copy(table_hbm.at[idx_slice(c)], bufs.at[b],
                                 gsem[b])

            def g_wait(c, b):
                pltpu.make_async_copy(table_hbm.at[idx_slice(c)], bufs.at[b],
                                      gsem[b]).wait()

            def s_start(c, b):
                pltpu.async_copy(bufs.at[b], out_slice(c), osem[b])

            def s_wait(c, b):
                pltpu.make_async_copy(bufs.at[b], out_slice(c),
                                      osem[b]).wait()

            g_start(0, 0)
            g_start(1, 1)
            g_wait(0, 0)
            s_start(0, 0)

            def body(g, carry):
                for u in range(2):
                    c = 1 + g * 2 + u
                    b = (1 + u) % 2
                    s_wait(c - 1, 1 - b)
                    g_start(c + 1, 1 - b)
                    g_wait(c, b)
                    s_start(c, b)
                return carry

            lax.fori_loop(0, (nch - 2) // 2, body, 0)
            c = nch - 1
            b = c % 2
            g_wait(c, b)
            s_start(c, b)
            s_wait(c - 1, 1 - b)
            s_wait(c, b)

        run(cidx_v, hand_hbm, hand_ch)
        run(xidx_v, ctx_hbm, ctx_ch)

    return gather


def kernel(hand_card_ids, hand_card_mask, hands_remaining, discards_remaining,
           hand_levels, rank_emb, suit_emb, proj_w, proj_b,
           run_ln_g, run_ln_b, hand_ln_g, hand_ln_b, level_emb):
    B, H = hand_card_ids.shape
    NT = hand_levels.shape[1]
    row = lambda v: v.astype(jnp.float32).reshape(1, D)
    table = _build_table(
        rank_emb.astype(jnp.float32), suit_emb.astype(jnp.float32),
        level_emb.astype(jnp.float32), proj_w.astype(jnp.float32).T,
        row(proj_b), row(run_ln_g), row(run_ln_b),
        row(hand_ln_g), row(hand_ln_b))

    table_rep = jnp.tile(table, (_NW, 1))
    ids = hand_card_ids.astype(jnp.int32)
    cidx = jnp.where(hand_card_mask, ids, _MASKED).astype(jnp.int32)
    cidx = cidx.reshape(B * H)
    cidx = cidx + _TROWS * (jnp.arange(B * H, dtype=jnp.int32)
                            // (B * H // _NW))
    run_idx = (_RUN0 + 4 * hands_remaining.astype(jnp.int32)
               + discards_remaining.astype(jnp.int32))
    xidx = jnp.concatenate(
        [hand_levels.astype(jnp.int32) + _LEVEL0, run_idx], axis=1)
    xidx = xidx.reshape(B * (NT + 1))
    xidx = xidx + _TROWS * (jnp.arange(B * (NT + 1), dtype=jnp.int32)
                            // (B * (NT + 1) // _NW))

    hand_flat, ctx_flat = _make_gather(B * H, B * (NT + 1))(
        table_rep, cidx, xidx)
    hand_toks = hand_flat.reshape(B, H, D)
    ctx_seq = ctx_flat.reshape(B, NT + 1, D)
    mask = hand_card_mask.astype(bool)
    ctx_mask = jnp.ones((B, NT + 1), dtype=bool)
    return hand_toks, mask, ctx_seq, ctx_mask


# speedup vs baseline: 6.1485x; 1.3072x over previous
import functools

import jax
import jax.numpy as jnp
from jax import lax
from jax.experimental import pallas as pl
from jax.experimental.pallas import tpu as pltpu
from jax.experimental.pallas import tpu_sc as plsc

D = 128
_EPS = 1e-5

_CARD0 = 0
_MASKED = 52
_LEVEL0 = 56
_RUN0 = 72
_TROWS = 96

_NC = 2
_NS = 16
_NW = _NC * _NS
_CH = 128
_CTX_G = 256


def _ln_rows(x, g, b):
    mu = jnp.mean(x, axis=-1, keepdims=True)
    var = jnp.mean((x - mu) ** 2, axis=-1, keepdims=True)
    return (x - mu) / jnp.sqrt(var + _EPS) * g + b


def _table_kernel(rank_ref, suit_ref, level_ref, pwt_ref, pb_ref,
                  rg_ref, rb_ref, hg_ref, hb_ref, out_ref, rep_ref):
    hg = hg_ref[0:1, :]
    hb = hb_ref[0:1, :]
    card = jnp.concatenate(
        [rank_ref[:, :] + suit_ref[s:s + 1, :] for s in range(4)], axis=0)
    card_ln = _ln_rows(card, hg, hb)
    masked = jnp.broadcast_to(hb, (4, D))
    level = level_ref[:, :]
    ii = lax.broadcasted_iota(jnp.int32, (20, D), 0)
    h = (ii // 4).astype(jnp.float32)
    d = (ii % 4).astype(jnp.float32)
    x = h * pwt_ref[0:1, :] + d * pwt_ref[1:2, :] + pb_ref[0:1, :]
    run_ln = _ln_rows(x, rg_ref[0:1, :], rb_ref[0:1, :])
    pad = jnp.zeros((4, D), jnp.float32)
    tbl = jnp.concatenate([card_ln, masked, level, run_ln, pad], axis=0)
    out_ref[:, :] = tbl
    for s in range(_NW):
        rep_ref[pl.ds(s * _TROWS, _TROWS), :] = tbl


def _build_table(rank_emb, suit_emb, level_emb, pwt, pb, rg, rb, hg, hb):
    return pl.pallas_call(
        _table_kernel,
        out_shape=(
            jax.ShapeDtypeStruct((_TROWS, D), jnp.float32),
            jax.ShapeDtypeStruct((_NW * _TROWS, D), jnp.float32),
        ),
    )(rank_emb, suit_emb, level_emb, pwt, pb, rg, rb, hg, hb)


def _ctx_kernel(xidx_ref, tbl_ref, out_ref):
    nt1 = xidx_ref.shape[1]
    idx = xidx_ref[:, :]
    oh = (idx[:, :, None]
          == lax.broadcasted_iota(jnp.int32, (_CTX_G, nt1, _TROWS), 2))
    oh = jnp.where(oh, 1.0, 0.0).reshape(_CTX_G * nt1, _TROWS)
    rows = jax.lax.dot_general(
        oh, tbl_ref[:, :], (((1,), (0,)), ((), ())),
        preferred_element_type=jnp.float32,
        precision=jax.lax.Precision.HIGHEST)
    out_ref[:, :, :] = rows.reshape(_CTX_G, nt1, D)


@functools.cache
def _make_ctx(B, nt1):
    return pl.pallas_call(
        _ctx_kernel,
        grid=(B // _CTX_G,),
        in_specs=[
            pl.BlockSpec((_CTX_G, nt1), lambda i: (i, 0)),
            pl.BlockSpec((_TROWS, D), lambda i: (0, 0)),
        ],
        out_specs=pl.BlockSpec((_CTX_G, nt1, D), lambda i: (i, 0, 0)),
        out_shape=jax.ShapeDtypeStruct((B, nt1, D), jnp.float32),
    )


@functools.cache
def _make_gather(hand_rows):
    hand_ch = hand_rows // (_NW * _CH)
    mesh = plsc.VectorSubcoreMesh(core_axis_name="c", subcore_axis_name="s")

    @functools.partial(
        pl.kernel,
        mesh=mesh,
        out_type=jax.ShapeDtypeStruct((hand_rows, D), jnp.float32),
        scratch_types=[
            pltpu.VMEM((hand_ch * _CH,), jnp.int32),
            pltpu.VMEM((2, _CH, D), jnp.float32),
            pltpu.SemaphoreType.DMA,
            pltpu.SemaphoreType.DMA,
            pltpu.SemaphoreType.DMA,
            pltpu.SemaphoreType.DMA,
        ],
    )
    def gather(table_hbm, cidx_hbm, hand_hbm, cidx_v, bufs, g0, g1, o0, o1):
        wid = lax.axis_index("s") * _NC + lax.axis_index("c")
        gsem = (g0, g1)
        osem = (o0, o1)
        pltpu.sync_copy(
            cidx_hbm.at[pl.ds(pl.multiple_of(wid * (hand_ch * _CH), 8),
                              hand_ch * _CH)], cidx_v)

        def run(idx_v, out_hbm, nch):
            base = wid * nch * _CH

            def out_slice(c):
                return out_hbm.at[
                    pl.ds(pl.multiple_of(base + c * _CH, 8), _CH)]

            def idx_slice(c):
                return idx_v.at[pl.ds(pl.multiple_of(c * _CH, 8), _CH)]

            def g_start(c, b):
                pltpu.async_copy(table_hbm.at[idx_slice(c)], bufs.at[b],
                                 gsem[b])

            def g_wait(c, b):
                pltpu.make_async_copy(table_hbm.at[idx_slice(c)], bufs.at[b],
                                      gsem[b]).wait()

            def s_start(c, b):
                pltpu.async_copy(bufs.at[b], out_slice(c), osem[b])

            def s_wait(c, b):
                pltpu.make_async_copy(bufs.at[b], out_slice(c),
                                      osem[b]).wait()

            g_start(0, 0)
            g_start(1, 1)
            g_wait(0, 0)
            s_start(0, 0)

            def body(g, carry):
                for u in range(2):
                    c = 1 + g * 2 + u
                    b = (1 + u) % 2
                    s_wait(c - 1, 1 - b)
                    g_start(c + 1, 1 - b)
                    g_wait(c, b)
                    s_start(c, b)
                return carry

            lax.fori_loop(0, (nch - 2) // 2, body, 0)
            c = nch - 1
            b = c % 2
            g_wait(c, b)
            s_start(c, b)
            s_wait(c - 1, 1 - b)
            s_wait(c, b)

        run(cidx_v, hand_hbm, hand_ch)

    return gather


def kernel(hand_card_ids, hand_card_mask, hands_remaining, discards_remaining,
           hand_levels, rank_emb, suit_emb, proj_w, proj_b,
           run_ln_g, run_ln_b, hand_ln_g, hand_ln_b, level_emb):
    B, H = hand_card_ids.shape
    NT = hand_levels.shape[1]
    row = lambda v: v.astype(jnp.float32).reshape(1, D)
    table, table_rep = _build_table(
        rank_emb.astype(jnp.float32), suit_emb.astype(jnp.float32),
        level_emb.astype(jnp.float32), proj_w.astype(jnp.float32).T,
        row(proj_b), row(run_ln_g), row(run_ln_b),
        row(hand_ln_g), row(hand_ln_b))

    ids = hand_card_ids.astype(jnp.int32)
    cidx = jnp.where(hand_card_mask, ids, _MASKED).astype(jnp.int32)
    cidx = cidx.reshape(B * H)
    cidx = cidx + _TROWS * (jnp.arange(B * H, dtype=jnp.int32)
                            // (B * H // _NW))
    run_idx = (_RUN0 + 4 * hands_remaining.astype(jnp.int32)
               + discards_remaining.astype(jnp.int32))
    xidx = jnp.concatenate(
        [hand_levels.astype(jnp.int32) + _LEVEL0, run_idx], axis=1)

    hand_flat = _make_gather(B * H)(table_rep, cidx)
    ctx_seq = _make_ctx(B, NT + 1)(xidx, table)
    hand_toks = hand_flat.reshape(B, H, D)
    mask = hand_card_mask.astype(bool)
    ctx_mask = jnp.ones((B, NT + 1), dtype=bool)
    return hand_toks, mask, ctx_seq, ctx_mask
